# baseline (speedup 1.0000x reference)
import functools

import jax
import jax.numpy as jnp
from jax import lax
from jax.experimental import pallas as pl
from jax.experimental.pallas import tpu as pltpu
from jax.experimental.pallas import tpu_sc as plsc

NC, NS, L = 2, 16, 16
B = 16384
BS = B // 2
CHUNK = B // NS
NBLK = CHUNK // L
ORD = B
D = 128
OSLAB = BS // (NC * NS)

_mesh = plsc.VectorSubcoreMesh(core_axis_name="c", subcore_axis_name="s",
                               num_cores=NC, num_subcores=NS)


@functools.partial(
    pl.kernel,
    out_type=(
        jax.ShapeDtypeStruct((BS * 4,), jnp.int32),
        jax.ShapeDtypeStruct((BS, D), jnp.float32),
    ),
    mesh=_mesh,
    scratch_types=[
        pltpu.VMEM((CHUNK,), jnp.int32),
        pltpu.VMEM((CHUNK,), jnp.int32),
        pltpu.VMEM((2 * CHUNK,), jnp.float32),
        pltpu.VMEM((L,), jnp.int32),
        pltpu.VMEM((NS, L), jnp.int32),
        pltpu.VMEM((NBLK // 8, 128), jnp.int32),
        pltpu.VMEM((NBLK // 8, 128), jnp.int32),
        pltpu.VMEM((NBLK // 2, 128), jnp.int32),
        pltpu.VMEM((NBLK // 2, 128), jnp.int32),
        pltpu.VMEM((128,), jnp.int32),
        pltpu.VMEM((128,), jnp.int32),
        pltpu.VMEM((128, D), jnp.float32),
        pltpu.VMEM((128, D), jnp.float32),
        pltpu.VMEM_SHARED((NS, L), jnp.int32),
        pltpu.VMEM_SHARED((ORD,), jnp.int32),
        pltpu.VMEM_SHARED((ORD * 4,), jnp.int32),
        pltpu.SemaphoreType.DMA,
        pltpu.SemaphoreType.DMA,
        pltpu.SemaphoreType.DMA,
    ],
    compiler_params=pltpu.CompilerParams(needs_layout_passes=False),
)
def _dsneg_kernel(fav_hbm, ret_hbm, wfl_hbm, emb_hbm,
                  packed_out, emb_out,
                  fav_v, ret_v, w_v, cnt1_v, cnt_v,
                  dest_v, src_v, dest4_v, pk_v, ordA_v, ordB_v,
                  embgA_v, embgB_v,
                  cnt_sh, ord_sh, pk_sh,
                  gsem, stsem, wsem):
    c = lax.axis_index("c")
    s = lax.axis_index("s")
    base = s * CHUNK

    pltpu.sync_copy(fav_hbm.at[pl.ds(base, CHUNK)], fav_v)
    pltpu.sync_copy(ret_hbm.at[pl.ds(base, CHUNK)], ret_v)
    pltpu.sync_copy(wfl_hbm.at[pl.ds(2 * base, 2 * CHUNK)], w_v)
    cnt = jnp.zeros((L,), jnp.int32)
    for b in range(NBLK):
        f = fav_v[pl.ds(b * L, L)]
        r = ret_v[pl.ds(b * L, L)]
        cnt = cnt + plsc.all_reduce_population_count((f == 1) | (r == 1))
    cnt1_v[...] = cnt
    pltpu.sync_copy(cnt1_v, cnt_sh.at[s])
    plsc.subcore_barrier()

    pltpu.sync_copy(cnt_sh, cnt_v)
    widx = lax.iota(jnp.int32, L)
    zero = jnp.zeros((L,), jnp.int32)
    c0 = plsc.load_gather(cnt_v, [widx, zero])
    npos = jnp.sum(c0)
    pos_base = jnp.sum(jnp.where(widx < s, c0, 0))
    neg_base = npos + base - pos_base

    npos_s = jnp.full((L,), npos, jnp.int32).astype(jnp.float32)
    den_s = jnp.float32(BS) - npos_s
    nw_s = jnp.where(den_s == 0.0, jnp.float32(0.0),
                     (jnp.float32(2 * BS) - npos_s) / den_s)
    pos_base_s = jnp.full((L,), pos_base, jnp.int32)
    neg_base_s = jnp.full((L,), neg_base, jnp.int32)
    lane = widx
    ones = jnp.ones((L,), jnp.float32)
    carry_p = jnp.zeros((L,), jnp.int32)
    copies = []
    for b in range(NBLK):
        f = fav_v[pl.ds(b * L, L)]
        r = ret_v[pl.ds(b * L, L)]
        m = (f == 1) | (r == 1)
        mi = m.astype(jnp.int32)
        excl = plsc.cumsum(mi) - mi
        posd = pos_base_s + carry_p + excl
        negd = neg_base_s + (b * L - carry_p) + (lane - excl)
        dest = jnp.where(m, posd, negd)
        r_hi, c_off = b // 8, (b % 8) * L
        dest_v[r_hi, pl.ds(c_off, L)] = dest
        src_v[r_hi, pl.ds(c_off, L)] = base + b * L + lane
        carry_p = carry_p + plsc.all_reduce_population_count(m)
        rows = lane + (b * L)
        w0 = plsc.load_gather(w_v, [rows * 2])
        w1 = plsc.load_gather(w_v, [rows * 2 + 1])
        factor = jnp.where(m, ones, nw_s)
        d0 = dest * 4
        vals = (f, r, plsc.bitcast(w0 * factor, jnp.int32),
                plsc.bitcast(w1 * factor, jnp.int32))
        for k in range(4):
            pk_v[k * 8 + r_hi, pl.ds(c_off, L)] = vals[k]
            dest4_v[k * 8 + r_hi, pl.ds(c_off, L)] = d0 + k
        if b % 8 == 7:
            copies.append(pltpu.async_copy(
                src_v.at[r_hi], ord_sh.at[dest_v.at[r_hi]], stsem))
            for k in range(4):
                j = k * 8 + r_hi
                copies.append(pltpu.async_copy(
                    pk_v.at[j], pk_sh.at[dest4_v.at[j]], stsem))
    for cp in copies:
        cp.wait()
    plsc.subcore_barrier()

    oid0 = (c * NS + s) * OSLAB
    pltpu.sync_copy(ord_sh.at[pl.ds(oid0, 128)], ordA_v)
    pltpu.sync_copy(ord_sh.at[pl.ds(oid0 + 128, 128)], ordB_v)
    cpA = pltpu.async_copy(emb_hbm.at[ordA_v], embgA_v, gsem)
    cpB = pltpu.async_copy(emb_hbm.at[ordB_v], embgB_v, gsem)
    pltpu.sync_copy(pk_sh.at[pl.ds(oid0 * 4, OSLAB * 4)],
                    packed_out.at[pl.ds(oid0 * 4, OSLAB * 4)])
    cpA.wait()
    wA = pltpu.async_copy(embgA_v, emb_out.at[pl.ds(oid0, 128), :], wsem)
    cpB.wait()
    wB = pltpu.async_copy(embgB_v, emb_out.at[pl.ds(oid0 + 128, 128), :], wsem)
    wA.wait()
    wB.wait()


def kernel(fav, retweet, embedding, weights):
    fav1 = fav.reshape(B)
    ret1 = retweet.reshape(B)
    wfl = weights.reshape(2 * B)
    packed_flat, embo = _dsneg_kernel(fav1, ret1, wfl, embedding)
    packed = packed_flat.reshape(BS, 4)
    out_fav = packed[:, 0:1]
    out_ret = packed[:, 1:2]
    out_w = lax.bitcast_convert_type(packed[:, 2:4], jnp.float32)
    return out_fav, out_ret, embo, out_w


# speedup vs baseline: 1.3305x; 1.3305x over previous
import functools

import jax
import jax.numpy as jnp
from jax import lax
from jax.experimental import pallas as pl
from jax.experimental.pallas import tpu as pltpu
from jax.experimental.pallas import tpu_sc as plsc

NC, NS, L = 2, 16, 16
B = 16384
BS = B // 2
CHUNK = B // NS
NBLK = CHUNK // L
ORD = B
D = 128
OSLAB = BS // (NC * NS)

_mesh = plsc.VectorSubcoreMesh(core_axis_name="c", subcore_axis_name="s",
                               num_cores=NC, num_subcores=NS)


@functools.partial(
    pl.kernel,
    out_type=(
        jax.ShapeDtypeStruct((BS,), jnp.int32),
        jax.ShapeDtypeStruct((BS,), jnp.int32),
        jax.ShapeDtypeStruct((BS, D), jnp.float32),
        jax.ShapeDtypeStruct((BS * 2,), jnp.float32),
    ),
    mesh=_mesh,
    scratch_types=[
        pltpu.VMEM((CHUNK,), jnp.int32),
        pltpu.VMEM((CHUNK,), jnp.int32),
        pltpu.VMEM((2 * CHUNK,), jnp.float32),
        pltpu.VMEM((L,), jnp.int32),
        pltpu.VMEM((NS, L), jnp.int32),
        pltpu.VMEM((NBLK // 8, 128), jnp.int32),
        pltpu.VMEM((NBLK // 8, 128), jnp.int32),
        pltpu.VMEM((NBLK // 4, 128), jnp.int32),
        pltpu.VMEM((NBLK // 4, 128), jnp.float32),
        pltpu.VMEM((NBLK // 4, 128), jnp.int32),
        pltpu.VMEM((128,), jnp.int32),
        pltpu.VMEM((128,), jnp.int32),
        pltpu.VMEM((128, D), jnp.float32),
        pltpu.VMEM((128, D), jnp.float32),
        pltpu.VMEM_SHARED((NS, L), jnp.int32),
        pltpu.VMEM_SHARED((ORD,), jnp.int32),
        pltpu.VMEM_SHARED((ORD,), jnp.int32),
        pltpu.VMEM_SHARED((ORD,), jnp.int32),
        pltpu.VMEM_SHARED((ORD * 2,), jnp.float32),
        pltpu.SemaphoreType.DMA,
        pltpu.SemaphoreType.DMA,
        pltpu.SemaphoreType.DMA,
    ],
    compiler_params=pltpu.CompilerParams(needs_layout_passes=False),
)
def _dsneg_kernel(fav_hbm, ret_hbm, wfl_hbm, emb_hbm,
                  fav_out, ret_out, emb_out, w_out,
                  fav_v, ret_v, w_v, cnt1_v, cnt_v,
                  dest_v, src_v, fr_v, wv_v, dw_v, ordA_v, ordB_v,
                  embgA_v, embgB_v,
                  cnt_sh, ord_sh, f_sh, r_sh, w_sh,
                  gsem, stsem, wsem):
    c = lax.axis_index("c")
    s = lax.axis_index("s")
    base = s * CHUNK

    pltpu.sync_copy(fav_hbm.at[pl.ds(base, CHUNK)], fav_v)
    pltpu.sync_copy(ret_hbm.at[pl.ds(base, CHUNK)], ret_v)
    pltpu.sync_copy(wfl_hbm.at[pl.ds(2 * base, 2 * CHUNK)], w_v)
    cnt = jnp.zeros((L,), jnp.int32)
    for b in range(NBLK):
        f = fav_v[pl.ds(b * L, L)]
        r = ret_v[pl.ds(b * L, L)]
        cnt = cnt + plsc.all_reduce_population_count((f == 1) | (r == 1))
    cnt1_v[...] = cnt
    pltpu.sync_copy(cnt1_v, cnt_sh.at[s])
    plsc.subcore_barrier()

    pltpu.sync_copy(cnt_sh, cnt_v)
    widx = lax.iota(jnp.int32, L)
    zero = jnp.zeros((L,), jnp.int32)
    c0 = plsc.load_gather(cnt_v, [widx, zero])
    npos = jnp.sum(c0)
    pos_base = jnp.sum(jnp.where(widx < s, c0, 0))
    neg_base = npos + base - pos_base

    npos_s = jnp.full((L,), npos, jnp.int32).astype(jnp.float32)
    den_s = jnp.float32(BS) - npos_s
    nw_s = jnp.where(den_s == 0.0, jnp.float32(0.0),
                     (jnp.float32(2 * BS) - npos_s) / den_s)
    pos_base_s = jnp.full((L,), pos_base, jnp.int32)
    neg_base_s = jnp.full((L,), neg_base, jnp.int32)
    lane = widx
    ones = jnp.ones((L,), jnp.float32)
    carry_p = jnp.zeros((L,), jnp.int32)
    copies = []
    for b in range(NBLK):
        f = fav_v[pl.ds(b * L, L)]
        r = ret_v[pl.ds(b * L, L)]
        m = (f == 1) | (r == 1)
        mi = m.astype(jnp.int32)
        excl = plsc.cumsum(mi) - mi
        posd = pos_base_s + carry_p + excl
        negd = neg_base_s + (b * L - carry_p) + (lane - excl)
        dest = jnp.where(m, posd, negd)
        r_hi, c_off = b // 8, (b % 8) * L
        dest_v[r_hi, pl.ds(c_off, L)] = dest
        src_v[r_hi, pl.ds(c_off, L)] = base + b * L + lane
        carry_p = carry_p + plsc.all_reduce_population_count(m)
        rows = lane + (b * L)
        w0 = plsc.load_gather(w_v, [rows * 2])
        w1 = plsc.load_gather(w_v, [rows * 2 + 1])
        factor = jnp.where(m, ones, nw_s)
        fr_v[r_hi, pl.ds(c_off, L)] = f
        fr_v[8 + r_hi, pl.ds(c_off, L)] = r
        wv_v[r_hi, pl.ds(c_off, L)] = w0 * factor
        wv_v[8 + r_hi, pl.ds(c_off, L)] = w1 * factor
        dw_v[r_hi, pl.ds(c_off, L)] = dest * 2
        dw_v[8 + r_hi, pl.ds(c_off, L)] = dest * 2 + 1
        if b % 8 == 7:
            copies.append(pltpu.async_copy(
                src_v.at[r_hi], ord_sh.at[dest_v.at[r_hi]], stsem))
            copies.append(pltpu.async_copy(
                fr_v.at[r_hi], f_sh.at[dest_v.at[r_hi]], stsem))
            copies.append(pltpu.async_copy(
                fr_v.at[8 + r_hi], r_sh.at[dest_v.at[r_hi]], stsem))
            copies.append(pltpu.async_copy(
                wv_v.at[r_hi], w_sh.at[dw_v.at[r_hi]], stsem))
            copies.append(pltpu.async_copy(
                wv_v.at[8 + r_hi], w_sh.at[dw_v.at[8 + r_hi]], stsem))
    for cp in copies:
        cp.wait()
    plsc.subcore_barrier()

    oid0 = (c * NS + s) * OSLAB
    pltpu.sync_copy(ord_sh.at[pl.ds(oid0, 128)], ordA_v)
    pltpu.sync_copy(ord_sh.at[pl.ds(oid0 + 128, 128)], ordB_v)
    cpA = pltpu.async_copy(emb_hbm.at[ordA_v], embgA_v, gsem)
    cpB = pltpu.async_copy(emb_hbm.at[ordB_v], embgB_v, gsem)
    pltpu.sync_copy(f_sh.at[pl.ds(oid0, OSLAB)],
                    fav_out.at[pl.ds(oid0, OSLAB)])
    pltpu.sync_copy(r_sh.at[pl.ds(oid0, OSLAB)],
                    ret_out.at[pl.ds(oid0, OSLAB)])
    pltpu.sync_copy(w_sh.at[pl.ds(oid0 * 2, OSLAB * 2)],
                    w_out.at[pl.ds(oid0 * 2, OSLAB * 2)])
    cpA.wait()
    wA = pltpu.async_copy(embgA_v, emb_out.at[pl.ds(oid0, 128), :], wsem)
    cpB.wait()
    wB = pltpu.async_copy(embgB_v, emb_out.at[pl.ds(oid0 + 128, 128), :], wsem)
    wA.wait()
    wB.wait()


def kernel(fav, retweet, embedding, weights):
    fav1 = fav.reshape(B)
    ret1 = retweet.reshape(B)
    wfl = weights.reshape(2 * B)
    favo, reto, embo, wo = _dsneg_kernel(fav1, ret1, wfl, embedding)
    return (favo.reshape(BS, 1), reto.reshape(BS, 1), embo,
            wo.reshape(BS, 2))
